# Initial kernel scaffold; baseline (speedup 1.0000x reference)
import jax
import jax.numpy as jnp
from jax.experimental import pallas as pl


def kernel(hidden_states, classifier_weight, e_score_correction_bias):
    raise NotImplementedError("write your pallas kernel here")


# speedup vs baseline: 1.3491x; 1.3491x over previous
import functools

import jax
import jax.numpy as jnp
from jax import lax
from jax.experimental import pallas as pl
from jax.experimental.pallas import tpu as pltpu

TOKENS = 8192
HIDDEN = 2048
EXPERTS = 64
TOPK = 8
SCALE = 2.5

BLK = 512
EPAD = 128


def _router_body(h_ref, w_ref, b_ref, w_out_ref, i_out_ref):
    h = h_ref[...]
    w = w_ref[...]
    logits = jnp.dot(h, w, preferred_element_type=jnp.float32)
    col = lax.broadcasted_iota(jnp.int32, (BLK, EPAD), 1)
    valid = col < EXPERTS
    neg = jnp.float32(-1e30)
    logits = jnp.where(valid, logits, neg)
    m = jnp.max(logits, axis=-1, keepdims=True)
    e = jnp.where(valid, jnp.exp(logits - m), 0.0)
    s = jnp.sum(e, axis=-1, keepdims=True)
    probs = e / s
    bias = b_ref[...]
    cur = jnp.where(valid, probs + bias, neg)
    for j in range(TOPK):
        mj = jnp.max(cur, axis=-1, keepdims=True)
        hit = cur == mj
        idx = jnp.min(jnp.where(hit, col, EPAD), axis=-1, keepdims=True)
        one = col == idx
        w_out_ref[:, j] = jnp.sum(jnp.where(one, probs, 0.0), axis=-1) * SCALE
        i_out_ref[:, j] = idx[:, 0]
        cur = jnp.where(one, neg, cur)


@jax.jit
def kernel(hidden_states, classifier_weight, e_score_correction_bias):
    wt = jnp.zeros((HIDDEN, EPAD), jnp.float32).at[:, :EXPERTS].set(
        classifier_weight.T)
    bias = jnp.full((1, EPAD), -1e30, jnp.float32).at[0, :EXPERTS].set(
        e_score_correction_bias)
    grid = TOKENS // BLK
    w_out, i_out = pl.pallas_call(
        _router_body,
        grid=(grid,),
        in_specs=[
            pl.BlockSpec((BLK, HIDDEN), lambda i: (i, 0)),
            pl.BlockSpec((HIDDEN, EPAD), lambda i: (0, 0)),
            pl.BlockSpec((1, EPAD), lambda i: (0, 0)),
        ],
        out_specs=[
            pl.BlockSpec((BLK, TOPK), lambda i: (i, 0)),
            pl.BlockSpec((BLK, TOPK), lambda i: (i, 0)),
        ],
        out_shape=[
            jax.ShapeDtypeStruct((TOKENS, TOPK), jnp.float32),
            jax.ShapeDtypeStruct((TOKENS, TOPK), jnp.int32),
        ],
    )(hidden_states, wt, bias)
    return w_out, i_out
